# baseline (device time: 869423 ns/iter reference)
import jax
import jax.numpy as jnp
from jax import lax
from jax.experimental import pallas as pl
from jax.experimental.pallas import tpu as pltpu

N_DEV = 8
M, K, N = 4096, 4096, 8192
CH = M // N_DEV
N2 = N // 2


def _allreduce_body(
    p_ref,
    out_ref,
    y_hbm,
    recv_hbm,
    pv,
    rvr, svr,
    rvl, svl,
    qvr, qvl,
    amine,
    amax_all,
    rs_send_r, rs_recv_r, ag_send_r, ag_recv_r,
    rs_send_l, rs_recv_l, ag_send_l, ag_recv_l,
    ax_send, ax_recv,
    csem, pfsem, qsem,
):
    d = lax.axis_index("i")
    right = lax.rem(d + 1, N_DEV)
    left = lax.rem(d - 1 + N_DEV, N_DEV)

    def _local(src, dst):
        cp = pltpu.make_async_copy(src, dst, csem)
        cp.start()
        cp.wait()

    def _prefetch(t, slot):
        cr = lax.rem(d - 1 - t + 2 * N_DEV, N_DEV)
        cl = lax.rem(d + 1 + t, N_DEV)
        pltpu.make_async_copy(
            p_ref.at[pl.ds(cr * CH, CH), :N2], pv.at[slot, 0], pfsem.at[0]
        ).start()
        pltpu.make_async_copy(
            p_ref.at[pl.ds(cl * CH, CH), N2:], pv.at[slot, 1], pfsem.at[1]
        ).start()

    def _wait_prefetch(slot):
        pltpu.make_async_copy(p_ref.at[pl.ds(0, CH), :N2], pv.at[slot, 0], pfsem.at[0]).wait()
        pltpu.make_async_copy(p_ref.at[pl.ds(0, CH), N2:], pv.at[slot, 1], pfsem.at[1]).wait()

    _prefetch(0, 0)
    prev = None
    for t in range(N_DEV - 1):
        slot = t % 2
        _wait_prefetch(slot)
        if t == 0:
            svr[...] = pv[slot, 0]
            svl[...] = pv[slot, 1]
        else:
            prev[0].wait()
            prev[1].wait()
            _local(recv_hbm.at[t - 1, :, :N2], rvr)
            _local(recv_hbm.at[t - 1, :, N2:], rvl)
            svr[...] = pv[slot, 0] + rvr[...]
            svl[...] = pv[slot, 1] + rvl[...]
        rdma_r = pltpu.make_async_remote_copy(
            src_ref=svr,
            dst_ref=recv_hbm.at[t, :, :N2],
            send_sem=rs_send_r.at[t],
            recv_sem=rs_recv_r.at[t],
            device_id=(right,),
            device_id_type=pl.DeviceIdType.MESH,
        )
        rdma_l = pltpu.make_async_remote_copy(
            src_ref=svl,
            dst_ref=recv_hbm.at[t, :, N2:],
            send_sem=rs_send_l.at[t],
            recv_sem=rs_recv_l.at[t],
            device_id=(left,),
            device_id_type=pl.DeviceIdType.MESH,
        )
        rdma_r.start()
        rdma_l.start()
        _prefetch(t + 1, (t + 1) % 2)
        prev = (rdma_r, rdma_l)

    prev[0].wait()
    prev[1].wait()
    slot = (N_DEV - 1) % 2
    _wait_prefetch(slot)
    _local(recv_hbm.at[N_DEV - 2, :, :N2], rvr)
    _local(recv_hbm.at[N_DEV - 2, :, N2:], rvl)
    svr[...] = pv[slot, 0] + rvr[...]
    svl[...] = pv[slot, 1] + rvl[...]
    _local(svr, y_hbm.at[pl.ds(d * CH, CH), :N2])
    _local(svl, y_hbm.at[pl.ds(d * CH, CH), N2:])

    m_mine = jnp.maximum(
        jnp.max(jnp.abs(svr[...].astype(jnp.float32))),
        jnp.max(jnp.abs(svl[...].astype(jnp.float32))),
    )
    amine[...] = jnp.full((8, 128), m_mine, jnp.float32)
    for k in range(1, N_DEV):
        pltpu.make_async_remote_copy(
            src_ref=amine,
            dst_ref=amax_all.at[d],
            send_sem=ax_send.at[k - 1],
            recv_sem=ax_recv.at[k - 1],
            device_id=(lax.rem(d + k, N_DEV),),
            device_id_type=pl.DeviceIdType.MESH,
        ).start()

    def _ag_start(t):
        cr = lax.rem(d - t + 2 * N_DEV, N_DEV)
        cl = lax.rem(d + t, N_DEV)
        rdma_r = pltpu.make_async_remote_copy(
            src_ref=y_hbm.at[pl.ds(cr * CH, CH), :N2],
            dst_ref=y_hbm.at[pl.ds(cr * CH, CH), :N2],
            send_sem=ag_send_r.at[t],
            recv_sem=ag_recv_r.at[t],
            device_id=(right,),
            device_id_type=pl.DeviceIdType.MESH,
        )
        rdma_l = pltpu.make_async_remote_copy(
            src_ref=y_hbm.at[pl.ds(cl * CH, CH), N2:],
            dst_ref=y_hbm.at[pl.ds(cl * CH, CH), N2:],
            send_sem=ag_send_l.at[t],
            recv_sem=ag_recv_l.at[t],
            device_id=(left,),
            device_id_type=pl.DeviceIdType.MESH,
        )
        rdma_r.start()
        rdma_l.start()
        return rdma_r, rdma_l

    ag_prev = _ag_start(0)

    amax_g = m_mine
    for k in range(1, N_DEV):
        src = lax.rem(d - k + 2 * N_DEV, N_DEV)
        pltpu.make_async_remote_copy(
            src_ref=amine,
            dst_ref=amax_all.at[src],
            send_sem=ax_send.at[k - 1],
            recv_sem=ax_recv.at[k - 1],
            device_id=(d,),
            device_id_type=pl.DeviceIdType.MESH,
        ).wait_recv()
        amax_g = jnp.maximum(amax_g, jnp.max(amax_all[pl.ds(src, 1)]))
    scale = amax_g / 127.0
    rinv = 1.0 / scale
    CH2 = CH // 2

    def _quant(bfr, bfl, row_r, row_l):
        for h in range(2):
            qvr[...] = jnp.clip(
                jnp.round(bfr[pl.ds(h * CH2, CH2), :].astype(jnp.float32) * rinv),
                -127.0, 127.0,
            ) * scale
            qvl[...] = jnp.clip(
                jnp.round(bfl[pl.ds(h * CH2, CH2), :].astype(jnp.float32) * rinv),
                -127.0, 127.0,
            ) * scale
            sr = pltpu.make_async_copy(
                qvr, out_ref.at[pl.ds(row_r + h * CH2, CH2), :N2], qsem.at[0]
            )
            sl = pltpu.make_async_copy(
                qvl, out_ref.at[pl.ds(row_l + h * CH2, CH2), N2:], qsem.at[1]
            )
            sr.start()
            sl.start()
            sr.wait()
            sl.wait()

    _quant(svr, svl, d * CH, d * CH)

    for t in range(1, N_DEV):
        ag_prev[0].wait()
        ag_prev[1].wait()
        if t < N_DEV - 1:
            ag_prev = _ag_start(t)
        cr = lax.rem(d - t + 2 * N_DEV, N_DEV)
        cl = lax.rem(d + t, N_DEV)
        _local(y_hbm.at[pl.ds(cr * CH, CH), :N2], rvr)
        _local(y_hbm.at[pl.ds(cl * CH, CH), N2:], rvl)
        _quant(rvr, rvl, cr * CH, cl * CH)

    for k in range(1, N_DEV):
        pltpu.make_async_remote_copy(
            src_ref=amine,
            dst_ref=amax_all.at[d],
            send_sem=ax_send.at[k - 1],
            recv_sem=ax_recv.at[k - 1],
            device_id=(lax.rem(d + k, N_DEV),),
            device_id_type=pl.DeviceIdType.MESH,
        ).wait_send()


def _ring_allreduce_quant(p):
    out, _, _ = pl.pallas_call(
        _allreduce_body,
        out_shape=(
            jax.ShapeDtypeStruct((M, N), jnp.float32),
            jax.ShapeDtypeStruct((M, N), jnp.bfloat16),
            jax.ShapeDtypeStruct((N_DEV - 1, CH, N), jnp.bfloat16),
        ),
        in_specs=[pl.BlockSpec(memory_space=pl.ANY)],
        out_specs=(
            pl.BlockSpec(memory_space=pl.ANY),
            pl.BlockSpec(memory_space=pl.ANY),
            pl.BlockSpec(memory_space=pl.ANY),
        ),
        scratch_shapes=[
            pltpu.MemorySpace.VMEM((2, 2, CH, N2), jnp.bfloat16),
            pltpu.MemorySpace.VMEM((CH, N2), jnp.bfloat16),
            pltpu.MemorySpace.VMEM((CH, N2), jnp.bfloat16),
            pltpu.MemorySpace.VMEM((CH, N2), jnp.bfloat16),
            pltpu.MemorySpace.VMEM((CH, N2), jnp.bfloat16),
            pltpu.MemorySpace.VMEM((CH // 2, N2), jnp.float32),
            pltpu.MemorySpace.VMEM((CH // 2, N2), jnp.float32),
            pltpu.MemorySpace.VMEM((8, 128), jnp.float32),
            pltpu.MemorySpace.VMEM((N_DEV, 8, 128), jnp.float32),
            pltpu.SemaphoreType.DMA((N_DEV - 1,)),
            pltpu.SemaphoreType.DMA((N_DEV - 1,)),
            pltpu.SemaphoreType.DMA((N_DEV - 1,)),
            pltpu.SemaphoreType.DMA((N_DEV - 1,)),
            pltpu.SemaphoreType.DMA((N_DEV - 1,)),
            pltpu.SemaphoreType.DMA((N_DEV - 1,)),
            pltpu.SemaphoreType.DMA((N_DEV - 1,)),
            pltpu.SemaphoreType.DMA((N_DEV - 1,)),
            pltpu.SemaphoreType.DMA((N_DEV - 1,)),
            pltpu.SemaphoreType.DMA((N_DEV - 1,)),
            pltpu.SemaphoreType.DMA,
            pltpu.SemaphoreType.DMA((2,)),
            pltpu.SemaphoreType.DMA((2,)),
        ],
        compiler_params=pltpu.CompilerParams(
            has_side_effects=True, vmem_limit_bytes=63 * 1024 * 1024
        ),
    )(p)
    return out


def kernel(x, w_mat):
    p = jnp.dot(x, w_mat, preferred_element_type=jnp.bfloat16)
    return _ring_allreduce_quant(p)


# device time: 843531 ns/iter; 1.0307x vs baseline; 1.0307x over previous
import jax
import jax.numpy as jnp
from jax import lax
from jax.experimental import pallas as pl
from jax.experimental.pallas import tpu as pltpu

N_DEV = 8
M, K, N = 4096, 4096, 8192
CH = M // N_DEV
N2 = N // 2


def _allreduce_body(
    x_ref,
    w_ref,
    out_ref,
    y_hbm,
    recv_hbm,
    pc,
    rvr, svr,
    rvl, svl,
    qvr, qvl,
    amine,
    amax_all,
    rs_send_r, rs_recv_r, ag_send_r, ag_recv_r,
    rs_send_l, rs_recv_l, ag_send_l, ag_recv_l,
    ax_send, ax_recv,
    csem, qsem,
):
    d = lax.axis_index("i")
    right = lax.rem(d + 1, N_DEV)
    left = lax.rem(d - 1 + N_DEV, N_DEV)

    def _local(src, dst):
        cp = pltpu.make_async_copy(src, dst, csem)
        cp.start()
        cp.wait()

    def _pcompute(t, slot):
        cr = lax.rem(d - 1 - t + 2 * N_DEV, N_DEV)
        cl = lax.rem(d + 1 + t, N_DEV)
        pc[slot, 0] = jnp.dot(
            x_ref[pl.ds(cr * CH, CH), :], w_ref[:, :N2],
            preferred_element_type=jnp.float32,
        ).astype(jnp.bfloat16)
        pc[slot, 1] = jnp.dot(
            x_ref[pl.ds(cl * CH, CH), :], w_ref[:, N2:],
            preferred_element_type=jnp.float32,
        ).astype(jnp.bfloat16)

    _pcompute(0, 0)
    prev = None
    for t in range(N_DEV - 1):
        slot = t % 2
        if t == 0:
            svr[...] = pc[slot, 0]
            svl[...] = pc[slot, 1]
        else:
            prev[0].wait()
            prev[1].wait()
            _local(recv_hbm.at[t - 1, :, :N2], rvr)
            _local(recv_hbm.at[t - 1, :, N2:], rvl)
            svr[...] = pc[slot, 0] + rvr[...]
            svl[...] = pc[slot, 1] + rvl[...]
        rdma_r = pltpu.make_async_remote_copy(
            src_ref=svr,
            dst_ref=recv_hbm.at[t, :, :N2],
            send_sem=rs_send_r.at[t],
            recv_sem=rs_recv_r.at[t],
            device_id=(right,),
            device_id_type=pl.DeviceIdType.MESH,
        )
        rdma_l = pltpu.make_async_remote_copy(
            src_ref=svl,
            dst_ref=recv_hbm.at[t, :, N2:],
            send_sem=rs_send_l.at[t],
            recv_sem=rs_recv_l.at[t],
            device_id=(left,),
            device_id_type=pl.DeviceIdType.MESH,
        )
        rdma_r.start()
        rdma_l.start()
        _pcompute(t + 1, (t + 1) % 2)
        prev = (rdma_r, rdma_l)

    prev[0].wait()
    prev[1].wait()
    slot = (N_DEV - 1) % 2
    _local(recv_hbm.at[N_DEV - 2, :, :N2], rvr)
    _local(recv_hbm.at[N_DEV - 2, :, N2:], rvl)
    svr[...] = pc[slot, 0] + rvr[...]
    svl[...] = pc[slot, 1] + rvl[...]
    _local(svr, y_hbm.at[pl.ds(d * CH, CH), :N2])
    _local(svl, y_hbm.at[pl.ds(d * CH, CH), N2:])

    m_mine = jnp.maximum(
        jnp.max(jnp.abs(svr[...].astype(jnp.float32))),
        jnp.max(jnp.abs(svl[...].astype(jnp.float32))),
    )
    amine[...] = jnp.full((8, 128), m_mine, jnp.float32)
    for k in range(1, N_DEV):
        pltpu.make_async_remote_copy(
            src_ref=amine,
            dst_ref=amax_all.at[d],
            send_sem=ax_send.at[k - 1],
            recv_sem=ax_recv.at[k - 1],
            device_id=(lax.rem(d + k, N_DEV),),
            device_id_type=pl.DeviceIdType.MESH,
        ).start()

    def _ag_start(t):
        cr = lax.rem(d - t + 2 * N_DEV, N_DEV)
        cl = lax.rem(d + t, N_DEV)
        rdma_r = pltpu.make_async_remote_copy(
            src_ref=y_hbm.at[pl.ds(cr * CH, CH), :N2],
            dst_ref=y_hbm.at[pl.ds(cr * CH, CH), :N2],
            send_sem=ag_send_r.at[t],
            recv_sem=ag_recv_r.at[t],
            device_id=(right,),
            device_id_type=pl.DeviceIdType.MESH,
        )
        rdma_l = pltpu.make_async_remote_copy(
            src_ref=y_hbm.at[pl.ds(cl * CH, CH), N2:],
            dst_ref=y_hbm.at[pl.ds(cl * CH, CH), N2:],
            send_sem=ag_send_l.at[t],
            recv_sem=ag_recv_l.at[t],
            device_id=(left,),
            device_id_type=pl.DeviceIdType.MESH,
        )
        rdma_r.start()
        rdma_l.start()
        return rdma_r, rdma_l

    ag_prev = _ag_start(0)

    amax_g = m_mine
    for k in range(1, N_DEV):
        src = lax.rem(d - k + 2 * N_DEV, N_DEV)
        pltpu.make_async_remote_copy(
            src_ref=amine,
            dst_ref=amax_all.at[src],
            send_sem=ax_send.at[k - 1],
            recv_sem=ax_recv.at[k - 1],
            device_id=(d,),
            device_id_type=pl.DeviceIdType.MESH,
        ).wait_recv()
        amax_g = jnp.maximum(amax_g, jnp.max(amax_all[pl.ds(src, 1)]))
    scale = amax_g / 127.0
    rinv = 1.0 / scale
    CH2 = CH // 2

    def _quant(bfr, bfl, row_r, row_l):
        for h in range(2):
            qvr[...] = jnp.clip(
                jnp.round(bfr[pl.ds(h * CH2, CH2), :].astype(jnp.float32) * rinv),
                -127.0, 127.0,
            ) * scale
            qvl[...] = jnp.clip(
                jnp.round(bfl[pl.ds(h * CH2, CH2), :].astype(jnp.float32) * rinv),
                -127.0, 127.0,
            ) * scale
            sr = pltpu.make_async_copy(
                qvr, out_ref.at[pl.ds(row_r + h * CH2, CH2), :N2], qsem.at[0]
            )
            sl = pltpu.make_async_copy(
                qvl, out_ref.at[pl.ds(row_l + h * CH2, CH2), N2:], qsem.at[1]
            )
            sr.start()
            sl.start()
            sr.wait()
            sl.wait()

    _quant(svr, svl, d * CH, d * CH)

    for t in range(1, N_DEV):
        ag_prev[0].wait()
        ag_prev[1].wait()
        if t < N_DEV - 1:
            ag_prev = _ag_start(t)
        cr = lax.rem(d - t + 2 * N_DEV, N_DEV)
        cl = lax.rem(d + t, N_DEV)
        _local(y_hbm.at[pl.ds(cr * CH, CH), :N2], rvr)
        _local(y_hbm.at[pl.ds(cl * CH, CH), N2:], rvl)
        _quant(rvr, rvl, cr * CH, cl * CH)

    for k in range(1, N_DEV):
        pltpu.make_async_remote_copy(
            src_ref=amine,
            dst_ref=amax_all.at[d],
            send_sem=ax_send.at[k - 1],
            recv_sem=ax_recv.at[k - 1],
            device_id=(lax.rem(d + k, N_DEV),),
            device_id_type=pl.DeviceIdType.MESH,
        ).wait_send()


def _ring_allreduce_quant(x, w_mat):
    out, _, _ = pl.pallas_call(
        _allreduce_body,
        out_shape=(
            jax.ShapeDtypeStruct((M, N), jnp.float32),
            jax.ShapeDtypeStruct((M, N), jnp.bfloat16),
            jax.ShapeDtypeStruct((N_DEV - 1, CH, N), jnp.bfloat16),
        ),
        in_specs=[
            pl.BlockSpec(memory_space=pltpu.MemorySpace.VMEM),
            pl.BlockSpec(memory_space=pltpu.MemorySpace.VMEM),
        ],
        out_specs=(
            pl.BlockSpec(memory_space=pl.ANY),
            pl.BlockSpec(memory_space=pl.ANY),
            pl.BlockSpec(memory_space=pl.ANY),
        ),
        scratch_shapes=[
            pltpu.MemorySpace.VMEM((2, 2, CH, N2), jnp.bfloat16),
            pltpu.MemorySpace.VMEM((CH, N2), jnp.bfloat16),
            pltpu.MemorySpace.VMEM((CH, N2), jnp.bfloat16),
            pltpu.MemorySpace.VMEM((CH, N2), jnp.bfloat16),
            pltpu.MemorySpace.VMEM((CH, N2), jnp.bfloat16),
            pltpu.MemorySpace.VMEM((CH // 2, N2), jnp.float32),
            pltpu.MemorySpace.VMEM((CH // 2, N2), jnp.float32),
            pltpu.MemorySpace.VMEM((8, 128), jnp.float32),
            pltpu.MemorySpace.VMEM((N_DEV, 8, 128), jnp.float32),
            pltpu.SemaphoreType.DMA((N_DEV - 1,)),
            pltpu.SemaphoreType.DMA((N_DEV - 1,)),
            pltpu.SemaphoreType.DMA((N_DEV - 1,)),
            pltpu.SemaphoreType.DMA((N_DEV - 1,)),
            pltpu.SemaphoreType.DMA((N_DEV - 1,)),
            pltpu.SemaphoreType.DMA((N_DEV - 1,)),
            pltpu.SemaphoreType.DMA((N_DEV - 1,)),
            pltpu.SemaphoreType.DMA((N_DEV - 1,)),
            pltpu.SemaphoreType.DMA((N_DEV - 1,)),
            pltpu.SemaphoreType.DMA((N_DEV - 1,)),
            pltpu.SemaphoreType.DMA,
            pltpu.SemaphoreType.DMA((2,)),
        ],
        compiler_params=pltpu.CompilerParams(
            has_side_effects=True, vmem_limit_bytes=63 * 1024 * 1024
        ),
    )(x, w_mat)
    return out


def kernel(x, w_mat):
    return _ring_allreduce_quant(
        x.astype(jnp.bfloat16), w_mat.astype(jnp.bfloat16)
    )
